# baseline (speedup 1.0000x reference)
import functools
import math

import jax
import jax.numpy as jnp
from jax import lax
from jax.experimental import pallas as pl
from jax.experimental.pallas import tpu as pltpu
from jax.experimental.pallas import tpu_sc as plsc

VOCAB = 1000000
D_MODEL = 64
SCALE = math.sqrt(D_MODEL)

B_TOTAL = 4096 * 200
NUM_WORKERS = 32
B_PER_W = B_TOTAL // NUM_WORKERS
G = 128
NGRP = B_PER_W // G
LANES = 16
VREGS_PER_ROW = D_MODEL // LANES


def _body(x_hbm, w_hbm, out_hbm, idx_v, rows_v, gsem):
    nc = 2
    wid = lax.axis_index("s") * nc + lax.axis_index("c")
    base = wid * B_PER_W

    pltpu.sync_copy(x_hbm.at[pl.ds(base, B_PER_W)], idx_v)

    def group(g, _):
        pltpu.async_copy(w_hbm.at[idx_v.at[pl.ds(g * G, G)]], rows_v, gsem).wait()

        def row(r, _):
            rref = rows_v.at[r]
            for j in range(VREGS_PER_ROW):
                sl = pl.ds(j * LANES, LANES)
                rref[sl] = rref[sl] * SCALE
            return 0

        lax.fori_loop(0, G, row, 0, unroll=2)

        pltpu.sync_copy(rows_v, out_hbm.at[pl.ds(base + g * G, G)])
        return 0

    lax.fori_loop(0, NGRP, group, 0)


@jax.jit
def _embed(x_flat, weight):
    mesh = plsc.VectorSubcoreMesh(core_axis_name="c", subcore_axis_name="s")
    kfn = pl.kernel(
        _body,
        mesh=mesh,
        out_type=jax.ShapeDtypeStruct((B_TOTAL, D_MODEL), jnp.float32),
        scratch_types=[
            pltpu.VMEM((B_PER_W,), jnp.int32),
            pltpu.VMEM((G, D_MODEL), jnp.float32),
            pltpu.SemaphoreType.DMA,
        ],
        compiler_params=pltpu.CompilerParams(use_tc_tiling_on_sc=False),
    )
    return kfn(x_flat, weight)


def kernel(x, weight):
    out = _embed(x.reshape(B_TOTAL), weight)
    return out.reshape(x.shape[0], x.shape[1], D_MODEL)


# speedup vs baseline: 1.0494x; 1.0494x over previous
import functools
import math

import jax
import jax.numpy as jnp
from jax import lax
from jax.experimental import pallas as pl
from jax.experimental.pallas import tpu as pltpu
from jax.experimental.pallas import tpu_sc as plsc

VOCAB = 1000000
D_MODEL = 64
SCALE = math.sqrt(D_MODEL)

B_TOTAL = 4096 * 200
NUM_WORKERS = 32
B_PER_W = B_TOTAL // NUM_WORKERS
G = 128
NGRP = B_PER_W // G
LANES = 16
VREGS_PER_ROW = D_MODEL // LANES

NBUF_I = 4
NBUF_O = 2
T_OUTER = NGRP // NBUF_I


def _scale_group(src, dst):
    def row(r, _):
        s = src.at[r]
        d = dst.at[r]
        for j in range(VREGS_PER_ROW):
            sl = pl.ds(j * LANES, LANES)
            d[sl] = s[sl] * SCALE
        return 0

    lax.fori_loop(0, G, row, 0, unroll=2)


def _body(x_hbm, w_hbm, out_hbm, idx_v, in_rows, out_rows, gsem, wsem):
    nc = 2
    wid = lax.axis_index("s") * nc + lax.axis_index("c")
    base = wid * B_PER_W

    pltpu.sync_copy(x_hbm.at[pl.ds(base, B_PER_W)], idx_v)

    def gather_start(g, bi):
        pltpu.async_copy(
            w_hbm.at[idx_v.at[pl.ds(g * G, G)]], in_rows.at[bi], gsem.at[bi])

    def gather_wait(bi):
        pltpu.make_async_copy(
            w_hbm.at[idx_v.at[pl.ds(0, G)]], in_rows.at[bi], gsem.at[bi]).wait()

    def write_start(g, bo):
        pltpu.async_copy(
            out_rows.at[bo], out_hbm.at[pl.ds(base + g * G, G)], wsem.at[bo])

    def write_wait(bo):
        pltpu.make_async_copy(
            out_rows.at[bo], out_hbm.at[pl.ds(base, G)], wsem.at[bo]).wait()

    for b in range(NBUF_I):
        gather_start(b, b)

    def step(t, _):
        for k in range(NBUF_I):
            g = t * NBUF_I + k
            bo = k % NBUF_O
            gather_wait(k)
            if k >= NBUF_O:
                write_wait(bo)
            else:
                @pl.when(t > 0)
                def _():
                    write_wait(bo)
            _scale_group(in_rows.at[k], out_rows.at[bo])
            write_start(g, bo)

            @pl.when(t < T_OUTER - 1)
            def _():
                gather_start(g + NBUF_I, k)
        return 0

    lax.fori_loop(0, T_OUTER, step, 0)

    for bo in range(NBUF_O):
        write_wait(bo)


@jax.jit
def _embed(x_flat, weight):
    mesh = plsc.VectorSubcoreMesh(core_axis_name="c", subcore_axis_name="s")
    kfn = pl.kernel(
        _body,
        mesh=mesh,
        out_type=jax.ShapeDtypeStruct((B_TOTAL, D_MODEL), jnp.float32),
        scratch_types=[
            pltpu.VMEM((B_PER_W,), jnp.int32),
            pltpu.VMEM((NBUF_I, G, D_MODEL), jnp.float32),
            pltpu.VMEM((NBUF_O, G, D_MODEL), jnp.float32),
            pltpu.SemaphoreType.DMA((NBUF_I,)),
            pltpu.SemaphoreType.DMA((NBUF_O,)),
        ],
        compiler_params=pltpu.CompilerParams(use_tc_tiling_on_sc=False),
    )
    return kfn(x_flat, weight)


def kernel(x, weight):
    out = _embed(x.reshape(B_TOTAL), weight)
    return out.reshape(x.shape[0], x.shape[1], D_MODEL)
